# Initial kernel scaffold; baseline (speedup 1.0000x reference)
import jax
import jax.numpy as jnp
from jax.experimental import pallas as pl


def kernel(x, W, b):
    raise NotImplementedError("write your pallas kernel here")


# speedup vs baseline: 1.8168x; 1.8168x over previous
import jax
import jax.numpy as jnp
from jax.experimental import pallas as pl
from jax.experimental.pallas import tpu as pltpu

_MAXR = 15


def _wsym_kernel(w_ref, wt_ref, o_ref):
    o_ref[...] = w_ref[...] + wt_ref[...].T


def _mm_kernel(a_ref, w_ref, o_ref):
    @pl.when(pl.program_id(2) == 0)
    def _():
        o_ref[...] = jnp.zeros_like(o_ref)

    o_ref[...] += jnp.dot(a_ref[...], w_ref[...],
                          preferred_element_type=jnp.float32)


def _fwd_kernel(x_ref, xw_ref, bv_ref, rad_ref, g_ref, y_ref, idx_ref, lf_ref):
    s = pl.program_id(1)
    bb, D = x_ref.shape
    bv = bv_ref[0:1, :]

    @pl.when(s == 0)
    def _():
        x = x_ref[...]
        y_ref[...] = x
        sx = jnp.sum(xw_ref[...] * x * 0.5 + x * bv, axis=1, keepdims=True)
        lf_ref[...] = jnp.broadcast_to(sx, lf_ref.shape)
        idx_ref[...] = jnp.zeros_like(idx_ref)

    cur = y_ref[...]
    gh = (xw_ref[...] + bv) * 0.5
    logits = (1.0 - 2.0 * cur) * gh
    z = logits + g_ref[0]
    lane = jax.lax.broadcasted_iota(jnp.int32, (bb, D), 1)
    zmx = jnp.max(z, axis=1, keepdims=True)
    idx = jnp.min(jnp.where(z == zmx, lane, D), axis=1, keepdims=True)
    onehot = lane == idx
    mx = jnp.max(logits, axis=1, keepdims=True)
    sh = logits - mx
    lse = jnp.log(jnp.sum(jnp.exp(sh), axis=1, keepdims=True))
    val = jnp.sum(jnp.where(onehot, sh, 0.0), axis=1, keepdims=True)
    maskb = rad_ref[:, 0:1] > s
    lf_ref[...] += jnp.broadcast_to((val - lse) * maskb.astype(jnp.float32),
                                    lf_ref.shape)
    y_ref[...] = jnp.where(onehot & maskb, 1.0 - cur, cur)
    li = jax.lax.broadcasted_iota(jnp.int32, idx_ref.shape, 1)
    idx_ref[...] = jnp.where(li == s, jnp.broadcast_to(idx, idx_ref.shape),
                             idx_ref[...])


def _bwd_kernel(x_ref, yw_ref, y_ref, bv_ref, rad_ref, idx_ref, lb_ref,
                st_ref):
    s = pl.program_id(1)
    bb, D = x_ref.shape
    bv = bv_ref[0:1, :]

    @pl.when(s == 0)
    def _():
        y = y_ref[...]
        st_ref[...] = x_ref[...]
        sy = jnp.sum(yw_ref[...] * y * 0.5 + y * bv, axis=1, keepdims=True)
        lb_ref[...] = jnp.broadcast_to(sy, lb_ref.shape)

    cur = st_ref[...]
    li = jax.lax.broadcasted_iota(jnp.int32, idx_ref.shape, 1)
    idx = jnp.sum(jnp.where(li == s, idx_ref[...], 0), axis=1, keepdims=True)
    lane = jax.lax.broadcasted_iota(jnp.int32, (bb, D), 1)
    onehot = lane == idx
    maskb = rad_ref[:, 0:1] > s
    cur = jnp.where(onehot & maskb, 1.0 - cur, cur)
    st_ref[...] = cur
    gh = (yw_ref[...] + bv) * 0.5
    logits = (1.0 - 2.0 * cur) * gh
    mx = jnp.max(logits, axis=1, keepdims=True)
    sh = logits - mx
    lse = jnp.log(jnp.sum(jnp.exp(sh), axis=1, keepdims=True))
    val = jnp.sum(jnp.where(onehot, sh, 0.0), axis=1, keepdims=True)
    lb_ref[...] += jnp.broadcast_to((val - lse) * maskb.astype(jnp.float32),
                                    lb_ref.shape)


def _accept_kernel(x_ref, y_ref, lf_ref, lb_ref, u_ref, o_ref):
    la = lb_ref[:, 0:1] - lf_ref[:, 0:1]
    acc = (jnp.exp(la) >= u_ref[:, 0:1]).astype(jnp.float32)
    o_ref[...] = y_ref[...] * acc + (1.0 - acc) * x_ref[...]


def kernel(x, W, b):
    B, D = x.shape
    f32 = jnp.float32
    key = jax.random.key(42)
    k_r, k_cat, k_acc = jax.random.split(key, 3)
    radius = jax.random.randint(k_r, (B, 1), 1, 16)
    G = jnp.stack([
        jax.random.gumbel(jax.random.fold_in(k_cat, s), (B, D), f32)
        for s in range(_MAXR)
    ])
    u = jax.random.uniform(k_acc, (B,), dtype=f32)

    rad128 = jnp.broadcast_to(radius, (B, 128))
    u128 = jnp.broadcast_to(u[:, None], (B, 128))
    bv = jnp.broadcast_to(b[None, :], (8, D))

    bt = 256
    Wsym = pl.pallas_call(
        _wsym_kernel,
        grid=(D // bt, D // bt),
        in_specs=[pl.BlockSpec((bt, bt), lambda i, j: (i, j)),
                  pl.BlockSpec((bt, bt), lambda i, j: (j, i))],
        out_specs=pl.BlockSpec((bt, bt), lambda i, j: (i, j)),
        out_shape=jax.ShapeDtypeStruct((D, D), f32),
    )(W, W)

    bm, bn, bk = 256, 256, 512

    def mm(a):
        return pl.pallas_call(
            _mm_kernel,
            grid=(B // bm, D // bn, D // bk),
            in_specs=[pl.BlockSpec((bm, bk), lambda i, j, k: (i, k)),
                      pl.BlockSpec((bk, bn), lambda i, j, k: (k, j))],
            out_specs=pl.BlockSpec((bm, bn), lambda i, j, k: (i, j)),
            out_shape=jax.ShapeDtypeStruct((B, D), f32),
            compiler_params=pltpu.CompilerParams(
                dimension_semantics=("parallel", "parallel", "arbitrary")),
        )(a, Wsym)

    xw = mm(x)

    bb = 256
    nb = B // bb
    row = lambda ib, s: (ib, 0)
    y, idxbuf, lf = pl.pallas_call(
        _fwd_kernel,
        grid=(nb, _MAXR),
        in_specs=[
            pl.BlockSpec((bb, D), row),
            pl.BlockSpec((bb, D), row),
            pl.BlockSpec((8, D), lambda ib, s: (0, 0)),
            pl.BlockSpec((bb, 128), row),
            pl.BlockSpec((1, bb, D), lambda ib, s: (s, ib, 0)),
        ],
        out_specs=[
            pl.BlockSpec((bb, D), row),
            pl.BlockSpec((bb, 128), row),
            pl.BlockSpec((bb, 128), row),
        ],
        out_shape=[jax.ShapeDtypeStruct((B, D), f32),
                   jax.ShapeDtypeStruct((B, 128), jnp.int32),
                   jax.ShapeDtypeStruct((B, 128), f32)],
        compiler_params=pltpu.CompilerParams(
            dimension_semantics=("arbitrary", "arbitrary")),
    )(x, xw, bv, rad128, G)

    yw = mm(y)

    lb = pl.pallas_call(
        _bwd_kernel,
        grid=(nb, _MAXR),
        in_specs=[
            pl.BlockSpec((bb, D), row),
            pl.BlockSpec((bb, D), row),
            pl.BlockSpec((bb, D), row),
            pl.BlockSpec((8, D), lambda ib, s: (0, 0)),
            pl.BlockSpec((bb, 128), row),
            pl.BlockSpec((bb, 128), row),
        ],
        out_specs=pl.BlockSpec((bb, 128), row),
        out_shape=jax.ShapeDtypeStruct((B, 128), f32),
        scratch_shapes=[pltpu.VMEM((bb, D), f32)],
        compiler_params=pltpu.CompilerParams(
            dimension_semantics=("arbitrary", "arbitrary")),
    )(x, yw, y, bv, rad128, idxbuf)

    new_x = pl.pallas_call(
        _accept_kernel,
        grid=(nb,),
        in_specs=[
            pl.BlockSpec((bb, D), lambda ib: (ib, 0)),
            pl.BlockSpec((bb, D), lambda ib: (ib, 0)),
            pl.BlockSpec((bb, 128), lambda ib: (ib, 0)),
            pl.BlockSpec((bb, 128), lambda ib: (ib, 0)),
            pl.BlockSpec((bb, 128), lambda ib: (ib, 0)),
        ],
        out_specs=pl.BlockSpec((bb, D), lambda ib: (ib, 0)),
        out_shape=jax.ShapeDtypeStruct((B, D), f32),
    )(x, y, lf, lb, u128)
    return new_x


# speedup vs baseline: 1.9111x; 1.0519x over previous
import jax
import jax.numpy as jnp
from jax.experimental import pallas as pl
from jax.experimental.pallas import tpu as pltpu

_MAXR = 15


def _wsym_kernel(w_ref, wt_ref, o_ref):
    o_ref[...] = w_ref[...] + wt_ref[...].T


def _mm_kernel(a_ref, w_ref, o_ref):
    @pl.when(pl.program_id(2) == 0)
    def _():
        o_ref[...] = jnp.zeros_like(o_ref)

    o_ref[...] += jnp.dot(a_ref[...], w_ref[...],
                          preferred_element_type=jnp.float32)


def _fwd_kernel(x_ref, xw_ref, bv_ref, rad_ref, g_ref, y_ref, idx_ref, lf_ref,
                sg_ref, s_ref):
    s = pl.program_id(1)
    bb, D = x_ref.shape

    @pl.when(s == 0)
    def _():
        x = x_ref[...]
        bv = bv_ref[0:1, :]
        y_ref[...] = x
        gh = (xw_ref[...] + bv) * 0.5
        sg = (1.0 - 2.0 * x) * gh
        sg_ref[...] = sg
        s0 = jnp.sum(jnp.exp(sg), axis=1, keepdims=True)
        s_ref[...] = jnp.broadcast_to(s0, s_ref.shape)
        sx = jnp.sum(xw_ref[...] * x * 0.5 + x * bv, axis=1, keepdims=True)
        lf_ref[...] = jnp.broadcast_to(sx, lf_ref.shape)
        idx_ref[...] = jnp.zeros_like(idx_ref)

    sg = sg_ref[...]
    z = sg + g_ref[0]
    lane = jax.lax.broadcasted_iota(jnp.int32, (bb, D), 1)
    zmx = jnp.max(z, axis=1, keepdims=True)
    idx = jnp.min(jnp.where(z == zmx, lane, D), axis=1, keepdims=True)
    onehot = lane == idx
    val = jnp.sum(jnp.where(onehot, sg, 0.0), axis=1, keepdims=True)
    maskb = rad_ref[:, 0:1] > s
    maskf = maskb.astype(jnp.float32)
    ssum = s_ref[:, 0:1]
    lf_ref[...] += jnp.broadcast_to((val - jnp.log(ssum)) * maskf,
                                    lf_ref.shape)
    flip = onehot & maskb
    cur = y_ref[...]
    y_ref[...] = jnp.where(flip, 1.0 - cur, cur)
    sg_ref[...] = jnp.where(flip, -sg, sg)
    snew = ssum + (jnp.exp(-val) - jnp.exp(val)) * maskf
    s_ref[...] = jnp.broadcast_to(snew, s_ref.shape)
    li = jax.lax.broadcasted_iota(jnp.int32, idx_ref.shape, 1)
    idx_ref[...] = jnp.where(li == s, jnp.broadcast_to(idx, idx_ref.shape),
                             idx_ref[...])


def _bwd_kernel(x_ref, yw_ref, y_ref, bv_ref, rad_ref, idx_ref, lb_ref,
                sg_ref, s_ref):
    s = pl.program_id(1)
    bb, D = x_ref.shape

    @pl.when(s == 0)
    def _():
        y = y_ref[...]
        bv = bv_ref[0:1, :]
        gh = (yw_ref[...] + bv) * 0.5
        sg = (1.0 - 2.0 * x_ref[...]) * gh
        sg_ref[...] = sg
        s0 = jnp.sum(jnp.exp(sg), axis=1, keepdims=True)
        s_ref[...] = jnp.broadcast_to(s0, s_ref.shape)
        sy = jnp.sum(yw_ref[...] * y * 0.5 + y * bv, axis=1, keepdims=True)
        lb_ref[...] = jnp.broadcast_to(sy, lb_ref.shape)

    li = jax.lax.broadcasted_iota(jnp.int32, idx_ref.shape, 1)
    idx = jnp.sum(jnp.where(li == s, idx_ref[...], 0), axis=1, keepdims=True)
    lane = jax.lax.broadcasted_iota(jnp.int32, (bb, D), 1)
    onehot = lane == idx
    maskb = rad_ref[:, 0:1] > s
    maskf = maskb.astype(jnp.float32)
    sg = sg_ref[...]
    l_old = jnp.sum(jnp.where(onehot, sg, 0.0), axis=1, keepdims=True)
    flip = onehot & maskb
    sg_ref[...] = jnp.where(flip, -sg, sg)
    ssum = s_ref[:, 0:1] + (jnp.exp(-l_old) - jnp.exp(l_old)) * maskf
    s_ref[...] = jnp.broadcast_to(ssum, s_ref.shape)
    lb_ref[...] += jnp.broadcast_to((-l_old - jnp.log(ssum)) * maskf,
                                    lb_ref.shape)


def _accept_kernel(x_ref, y_ref, lf_ref, lb_ref, u_ref, o_ref):
    la = lb_ref[:, 0:1] - lf_ref[:, 0:1]
    acc = (jnp.exp(la) >= u_ref[:, 0:1]).astype(jnp.float32)
    o_ref[...] = y_ref[...] * acc + (1.0 - acc) * x_ref[...]


def kernel(x, W, b):
    B, D = x.shape
    f32 = jnp.float32
    key = jax.random.key(42)
    k_r, k_cat, k_acc = jax.random.split(key, 3)
    radius = jax.random.randint(k_r, (B, 1), 1, 16)
    G = jnp.stack([
        jax.random.gumbel(jax.random.fold_in(k_cat, s), (B, D), f32)
        for s in range(_MAXR)
    ])
    u = jax.random.uniform(k_acc, (B,), dtype=f32)

    rad128 = jnp.broadcast_to(radius, (B, 128))
    u128 = jnp.broadcast_to(u[:, None], (B, 128))
    bv = jnp.broadcast_to(b[None, :], (8, D))

    bt = 256
    Wsym = pl.pallas_call(
        _wsym_kernel,
        grid=(D // bt, D // bt),
        in_specs=[pl.BlockSpec((bt, bt), lambda i, j: (i, j)),
                  pl.BlockSpec((bt, bt), lambda i, j: (j, i))],
        out_specs=pl.BlockSpec((bt, bt), lambda i, j: (i, j)),
        out_shape=jax.ShapeDtypeStruct((D, D), f32),
    )(W, W)

    bm, bn, bk = 256, 256, 512

    def mm(a):
        return pl.pallas_call(
            _mm_kernel,
            grid=(B // bm, D // bn, D // bk),
            in_specs=[pl.BlockSpec((bm, bk), lambda i, j, k: (i, k)),
                      pl.BlockSpec((bk, bn), lambda i, j, k: (k, j))],
            out_specs=pl.BlockSpec((bm, bn), lambda i, j, k: (i, j)),
            out_shape=jax.ShapeDtypeStruct((B, D), f32),
            compiler_params=pltpu.CompilerParams(
                dimension_semantics=("parallel", "parallel", "arbitrary")),
        )(a, Wsym)

    xw = mm(x)

    bb = 256
    nb = B // bb
    row = lambda ib, s: (ib, 0)
    y, idxbuf, lf = pl.pallas_call(
        _fwd_kernel,
        grid=(nb, _MAXR),
        in_specs=[
            pl.BlockSpec((bb, D), row),
            pl.BlockSpec((bb, D), row),
            pl.BlockSpec((8, D), lambda ib, s: (0, 0)),
            pl.BlockSpec((bb, 128), row),
            pl.BlockSpec((1, bb, D), lambda ib, s: (s, ib, 0)),
        ],
        out_specs=[
            pl.BlockSpec((bb, D), row),
            pl.BlockSpec((bb, 128), row),
            pl.BlockSpec((bb, 128), row),
        ],
        out_shape=[jax.ShapeDtypeStruct((B, D), f32),
                   jax.ShapeDtypeStruct((B, 128), jnp.int32),
                   jax.ShapeDtypeStruct((B, 128), f32)],
        scratch_shapes=[pltpu.VMEM((bb, D), f32),
                        pltpu.VMEM((bb, 128), f32)],
        compiler_params=pltpu.CompilerParams(
            dimension_semantics=("arbitrary", "arbitrary")),
    )(x, xw, bv, rad128, G)

    yw = mm(y)

    lb = pl.pallas_call(
        _bwd_kernel,
        grid=(nb, _MAXR),
        in_specs=[
            pl.BlockSpec((bb, D), row),
            pl.BlockSpec((bb, D), row),
            pl.BlockSpec((bb, D), row),
            pl.BlockSpec((8, D), lambda ib, s: (0, 0)),
            pl.BlockSpec((bb, 128), row),
            pl.BlockSpec((bb, 128), row),
        ],
        out_specs=pl.BlockSpec((bb, 128), row),
        out_shape=jax.ShapeDtypeStruct((B, 128), f32),
        scratch_shapes=[pltpu.VMEM((bb, D), f32),
                        pltpu.VMEM((bb, 128), f32)],
        compiler_params=pltpu.CompilerParams(
            dimension_semantics=("arbitrary", "arbitrary")),
    )(x, yw, y, bv, rad128, idxbuf)

    new_x = pl.pallas_call(
        _accept_kernel,
        grid=(nb,),
        in_specs=[
            pl.BlockSpec((bb, D), lambda ib: (ib, 0)),
            pl.BlockSpec((bb, D), lambda ib: (ib, 0)),
            pl.BlockSpec((bb, 128), lambda ib: (ib, 0)),
            pl.BlockSpec((bb, 128), lambda ib: (ib, 0)),
            pl.BlockSpec((bb, 128), lambda ib: (ib, 0)),
        ],
        out_specs=pl.BlockSpec((bb, D), lambda ib: (ib, 0)),
        out_shape=jax.ShapeDtypeStruct((B, D), f32),
    )(x, y, lf, lb, u128)
    return new_x


# speedup vs baseline: 1.9644x; 1.0279x over previous
import jax
import jax.numpy as jnp
from jax.experimental import pallas as pl
from jax.experimental.pallas import tpu as pltpu

_MAXR = 15


def _wsym_kernel(w_ref, wt_ref, o_ref):
    o_ref[...] = w_ref[...] + wt_ref[...].T


def _mm_kernel(a_ref, w_ref, o_ref):
    @pl.when(pl.program_id(2) == 0)
    def _():
        o_ref[...] = jnp.zeros_like(o_ref)

    o_ref[...] += jnp.dot(a_ref[...], w_ref[...],
                          preferred_element_type=jnp.float32)


def _fwd_kernel(karr_ref, x_ref, xw_ref, bv_ref, rad_ref, y_ref, idx_ref,
                lf_ref, sg_ref):
    ib = pl.program_id(0)
    bb, D = x_ref.shape
    x = x_ref[...]
    bv = bv_ref[0:1, :]
    y_ref[...] = x
    gh = (xw_ref[...] + bv) * 0.5
    sg0 = (1.0 - 2.0 * x) * gh
    sg_ref[...] = sg0
    s0 = jnp.sum(jnp.exp(sg0), axis=1, keepdims=True)
    sx = jnp.sum(xw_ref[...] * x * 0.5 + x * bv, axis=1, keepdims=True)
    idx_ref[...] = jnp.zeros_like(idx_ref)
    lane = jax.lax.broadcasted_iota(jnp.int32, (bb, D), 1)
    rowi = jax.lax.broadcasted_iota(jnp.int32, (bb, D), 0)
    cnt = (rowi * D + lane + ib * (bb * D)).astype(jnp.uint32)
    li = jax.lax.broadcasted_iota(jnp.int32, idx_ref.shape, 1)
    rad = rad_ref[:, 0:1]
    rots = ((13, 15, 26, 6), (17, 29, 16, 24))

    def body(s, carry):
        ssum, lf = carry
        ks0 = karr_ref[0, s]
        ks1 = karr_ref[1, s]
        ks2 = karr_ref[2, s]
        ksl = (ks0, ks1, ks2)
        h0 = ks0
        h1 = cnt + ks1
        for i in range(5):
            for r in rots[i % 2]:
                h0 = h0 + h1
                h1 = (h1 << jnp.uint32(r)) | (h1 >> jnp.uint32(32 - r))
                h1 = h0 ^ h1
            h0 = h0 + ksl[(i + 1) % 3]
            h1 = h1 + ksl[(i + 2) % 3] + jnp.uint32(i + 1)
        bits = h0 ^ h1
        fb = (bits >> jnp.uint32(9)) | jnp.uint32(0x3F800000)
        f = jax.lax.bitcast_convert_type(fb, jnp.float32) - 1.0
        tiny = jnp.float32(jnp.finfo(jnp.float32).tiny)
        uu = jnp.maximum(tiny, f * (jnp.float32(1.0) - tiny) + tiny)
        g = -jnp.log(-jnp.log(uu))

        sg = sg_ref[...]
        z = sg + g
        zmx = jnp.max(z, axis=1, keepdims=True)
        idx = jnp.min(jnp.where(z == zmx, lane, D), axis=1, keepdims=True)
        onehot = lane == idx
        val = jnp.sum(jnp.where(onehot, sg, 0.0), axis=1, keepdims=True)
        maskb = rad > s
        maskf = maskb.astype(jnp.float32)
        lf = lf + (val - jnp.log(ssum)) * maskf
        flip = onehot & maskb
        cur = y_ref[...]
        y_ref[...] = jnp.where(flip, 1.0 - cur, cur)
        sg_ref[...] = jnp.where(flip, -sg, sg)
        ssum = ssum + (jnp.exp(-val) - jnp.exp(val)) * maskf
        idx_ref[...] = jnp.where(li == s, jnp.broadcast_to(idx, idx_ref.shape),
                                 idx_ref[...])
        return ssum, lf

    _, lf = jax.lax.fori_loop(0, _MAXR, body, (s0, sx))
    lf_ref[...] = jnp.broadcast_to(lf, lf_ref.shape)


def _bwd_kernel(x_ref, yw_ref, y_ref, bv_ref, rad_ref, idx_ref, lf_ref,
                u_ref, o_ref, sg_ref):
    bb, D = x_ref.shape
    x = x_ref[...]
    y = y_ref[...]
    bv = bv_ref[0:1, :]
    gh = (yw_ref[...] + bv) * 0.5
    sg0 = (1.0 - 2.0 * x) * gh
    sg_ref[...] = sg0
    s0 = jnp.sum(jnp.exp(sg0), axis=1, keepdims=True)
    sy = jnp.sum(yw_ref[...] * y * 0.5 + y * bv, axis=1, keepdims=True)
    idxb = idx_ref[...]
    lane = jax.lax.broadcasted_iota(jnp.int32, (bb, D), 1)
    li = jax.lax.broadcasted_iota(jnp.int32, idxb.shape, 1)
    rad = rad_ref[:, 0:1]

    def body(s, carry):
        ssum, lb = carry
        idx = jnp.sum(jnp.where(li == s, idxb, 0), axis=1, keepdims=True)
        onehot = lane == idx
        maskb = rad > s
        maskf = maskb.astype(jnp.float32)
        sg = sg_ref[...]
        l_old = jnp.sum(jnp.where(onehot, sg, 0.0), axis=1, keepdims=True)
        sg_ref[...] = jnp.where(onehot & maskb, -sg, sg)
        ssum = ssum + (jnp.exp(-l_old) - jnp.exp(l_old)) * maskf
        lb = lb + (-l_old - jnp.log(ssum)) * maskf
        return ssum, lb

    _, lb = jax.lax.fori_loop(0, _MAXR, body, (s0, sy))
    la = lb - lf_ref[:, 0:1]
    acc = (jnp.exp(la) >= u_ref[:, 0:1]).astype(jnp.float32)
    o_ref[...] = y * acc + (1.0 - acc) * x


def kernel(x, W, b):
    B, D = x.shape
    f32 = jnp.float32
    key = jax.random.key(42)
    k_r, k_cat, k_acc = jax.random.split(key, 3)
    radius = jax.random.randint(k_r, (B, 1), 1, 16)
    u = jax.random.uniform(k_acc, (B,), dtype=f32)
    kd = jnp.stack([jax.random.key_data(jax.random.fold_in(k_cat, s))
                    for s in range(_MAXR)], axis=1)
    karr = jnp.concatenate(
        [kd, (kd[0:1] ^ kd[1:2] ^ jnp.uint32(0x1BD11BDA))])

    rad128 = jnp.broadcast_to(radius, (B, 128))
    u128 = jnp.broadcast_to(u[:, None], (B, 128))
    bv = jnp.broadcast_to(b[None, :], (8, D))

    bt = 256
    Wsym = pl.pallas_call(
        _wsym_kernel,
        grid=(D // bt, D // bt),
        in_specs=[pl.BlockSpec((bt, bt), lambda i, j: (i, j)),
                  pl.BlockSpec((bt, bt), lambda i, j: (j, i))],
        out_specs=pl.BlockSpec((bt, bt), lambda i, j: (i, j)),
        out_shape=jax.ShapeDtypeStruct((D, D), f32),
    )(W, W)

    bm, bn, bk = 256, 256, 512

    def mm(a):
        return pl.pallas_call(
            _mm_kernel,
            grid=(B // bm, D // bn, D // bk),
            in_specs=[pl.BlockSpec((bm, bk), lambda i, j, k: (i, k)),
                      pl.BlockSpec((bk, bn), lambda i, j, k: (k, j))],
            out_specs=pl.BlockSpec((bm, bn), lambda i, j, k: (i, j)),
            out_shape=jax.ShapeDtypeStruct((B, D), f32),
            compiler_params=pltpu.CompilerParams(
                dimension_semantics=("parallel", "parallel", "arbitrary")),
        )(a, Wsym)

    xw = mm(x)

    bb = 256
    nb = B // bb
    row = lambda ib: (ib, 0)
    y, idxbuf, lf = pl.pallas_call(
        _fwd_kernel,
        grid=(nb,),
        in_specs=[
            pl.BlockSpec(memory_space=pltpu.SMEM),
            pl.BlockSpec((bb, D), row),
            pl.BlockSpec((bb, D), row),
            pl.BlockSpec((8, D), lambda ib: (0, 0)),
            pl.BlockSpec((bb, 128), row),
        ],
        out_specs=[
            pl.BlockSpec((bb, D), row),
            pl.BlockSpec((bb, 128), row),
            pl.BlockSpec((bb, 128), row),
        ],
        out_shape=[jax.ShapeDtypeStruct((B, D), f32),
                   jax.ShapeDtypeStruct((B, 128), jnp.int32),
                   jax.ShapeDtypeStruct((B, 128), f32)],
        scratch_shapes=[pltpu.VMEM((bb, D), f32)],
        compiler_params=pltpu.CompilerParams(
            dimension_semantics=("arbitrary",)),
    )(karr, x, xw, bv, rad128)

    yw = mm(y)

    new_x = pl.pallas_call(
        _bwd_kernel,
        grid=(nb,),
        in_specs=[
            pl.BlockSpec((bb, D), row),
            pl.BlockSpec((bb, D), row),
            pl.BlockSpec((bb, D), row),
            pl.BlockSpec((8, D), lambda ib: (0, 0)),
            pl.BlockSpec((bb, 128), row),
            pl.BlockSpec((bb, 128), row),
            pl.BlockSpec((bb, 128), row),
            pl.BlockSpec((bb, 128), row),
        ],
        out_specs=pl.BlockSpec((bb, D), row),
        out_shape=jax.ShapeDtypeStruct((B, D), f32),
        scratch_shapes=[pltpu.VMEM((bb, D), f32)],
        compiler_params=pltpu.CompilerParams(
            dimension_semantics=("arbitrary",)),
    )(x, yw, y, bv, rad128, idxbuf, lf, u128)
    return new_x


# speedup vs baseline: 2.0047x; 1.0205x over previous
import jax
import jax.numpy as jnp
from jax.experimental import pallas as pl
from jax.experimental.pallas import tpu as pltpu

_MAXR = 15


def _wsym_kernel(w_ref, wt_ref, o_ref):
    o_ref[...] = w_ref[...] + wt_ref[...].T


def _mm_kernel(a_ref, w_ref, o_ref):
    @pl.when(pl.program_id(2) == 0)
    def _():
        o_ref[...] = jnp.zeros_like(o_ref)

    o_ref[...] += jnp.dot(a_ref[...], w_ref[...],
                          preferred_element_type=jnp.float32)


def _fwd_kernel(karr_ref, x_ref, xw_ref, bv_ref, rad_ref, y_ref, idx_ref,
                lf_ref, sg_ref):
    ib = pl.program_id(0)
    bb, D = x_ref.shape
    x = x_ref[...]
    bv = bv_ref[0:1, :]
    gh = (xw_ref[...] + bv) * 0.5
    sg0 = (1.0 - 2.0 * x) * gh
    sg_ref[...] = sg0
    s0 = jnp.sum(jnp.exp(sg0), axis=1, keepdims=True)
    sx = jnp.sum(xw_ref[...] * x * 0.5 + x * bv, axis=1, keepdims=True)
    idx_ref[...] = jnp.zeros_like(idx_ref)
    lane = jax.lax.broadcasted_iota(jnp.int32, (bb, D), 1)
    rowi = jax.lax.broadcasted_iota(jnp.int32, (bb, D), 0)
    cnt = (rowi * D + lane + ib * (bb * D)).astype(jnp.uint32)
    li = jax.lax.broadcasted_iota(jnp.int32, idx_ref.shape, 1)
    rad = rad_ref[:, 0:1]
    rots = ((13, 15, 26, 6), (17, 29, 16, 24))

    def body(s, carry):
        ssum, lf = carry
        ks0 = karr_ref[0, s]
        ks1 = karr_ref[1, s]
        ks2 = karr_ref[2, s]
        ksl = (ks0, ks1, ks2)
        h0 = ks0
        h1 = cnt + ks1
        for i in range(5):
            for r in rots[i % 2]:
                h0 = h0 + h1
                h1 = (h1 << jnp.uint32(r)) | (h1 >> jnp.uint32(32 - r))
                h1 = h0 ^ h1
            h0 = h0 + ksl[(i + 1) % 3]
            h1 = h1 + ksl[(i + 2) % 3] + jnp.uint32(i + 1)
        bits = h0 ^ h1
        fb = (bits >> jnp.uint32(9)) | jnp.uint32(0x3F800000)
        f = jax.lax.bitcast_convert_type(fb, jnp.float32) - 1.0
        tiny = jnp.float32(jnp.finfo(jnp.float32).tiny)
        uu = jnp.maximum(tiny, f * (jnp.float32(1.0) - tiny) + tiny)
        g = -jnp.log(-jnp.log(uu))

        sg = sg_ref[...]
        z = sg + g
        zmx = jnp.max(z, axis=1, keepdims=True)
        idx = jnp.min(jnp.where(z == zmx, lane, D), axis=1, keepdims=True)
        onehot = lane == idx
        val = jnp.sum(jnp.where(onehot, sg, 0.0), axis=1, keepdims=True)
        maskb = rad > s
        maskf = maskb.astype(jnp.float32)
        lf = lf + (val - jnp.log(ssum)) * maskf
        flip = onehot & maskb
        sg_ref[...] = jnp.where(flip, -sg, sg)
        ssum = ssum + (jnp.exp(-val) - jnp.exp(val)) * maskf
        idx_ref[...] = jnp.where(li == s, jnp.broadcast_to(idx, idx_ref.shape),
                                 idx_ref[...])
        return ssum, lf

    _, lf = jax.lax.fori_loop(0, _MAXR, body, (s0, sx))
    lf_ref[...] = jnp.broadcast_to(lf, lf_ref.shape)
    sgi = jax.lax.bitcast_convert_type(sg_ref[...], jnp.uint32)
    ghi = jax.lax.bitcast_convert_type((xw_ref[...] + bv) * 0.5, jnp.uint32)
    y_ref[...] = ((sgi ^ ghi) >> jnp.uint32(31)).astype(jnp.float32)


def _bwd_kernel(x_ref, yw_ref, y_ref, bv_ref, rad_ref, idx_ref, lf_ref,
                u_ref, o_ref, sg_ref):
    bb, D = x_ref.shape
    x = x_ref[...]
    y = y_ref[...]
    bv = bv_ref[0:1, :]
    gh = (yw_ref[...] + bv) * 0.5
    sg0 = (1.0 - 2.0 * x) * gh
    sg_ref[...] = sg0
    s0 = jnp.sum(jnp.exp(sg0), axis=1, keepdims=True)
    sy = jnp.sum(yw_ref[...] * y * 0.5 + y * bv, axis=1, keepdims=True)
    idxb = idx_ref[...]
    lane = jax.lax.broadcasted_iota(jnp.int32, (bb, D), 1)
    li = jax.lax.broadcasted_iota(jnp.int32, idxb.shape, 1)
    rad = rad_ref[:, 0:1]

    def body(s, carry):
        ssum, lb = carry
        idx = jnp.sum(jnp.where(li == s, idxb, 0), axis=1, keepdims=True)
        onehot = lane == idx
        maskb = rad > s
        maskf = maskb.astype(jnp.float32)
        sg = sg_ref[...]
        l_old = jnp.sum(jnp.where(onehot, sg, 0.0), axis=1, keepdims=True)
        sg_ref[...] = jnp.where(onehot & maskb, -sg, sg)
        ssum = ssum + (jnp.exp(-l_old) - jnp.exp(l_old)) * maskf
        lb = lb + (-l_old - jnp.log(ssum)) * maskf
        return ssum, lb

    _, lb = jax.lax.fori_loop(0, _MAXR, body, (s0, sy))
    la = lb - lf_ref[:, 0:1]
    acc = (jnp.exp(la) >= u_ref[:, 0:1]).astype(jnp.float32)
    o_ref[...] = y * acc + (1.0 - acc) * x


def kernel(x, W, b):
    B, D = x.shape
    f32 = jnp.float32
    key = jax.random.key(42)
    k_r, k_cat, k_acc = jax.random.split(key, 3)
    radius = jax.random.randint(k_r, (B, 1), 1, 16)
    u = jax.random.uniform(k_acc, (B,), dtype=f32)
    kd = jnp.stack([jax.random.key_data(jax.random.fold_in(k_cat, s))
                    for s in range(_MAXR)], axis=1)
    karr = jnp.concatenate(
        [kd, (kd[0:1] ^ kd[1:2] ^ jnp.uint32(0x1BD11BDA))])

    rad128 = jnp.broadcast_to(radius, (B, 128))
    u128 = jnp.broadcast_to(u[:, None], (B, 128))
    bv = jnp.broadcast_to(b[None, :], (8, D))

    bt = 256
    Wsym = pl.pallas_call(
        _wsym_kernel,
        grid=(D // bt, D // bt),
        in_specs=[pl.BlockSpec((bt, bt), lambda i, j: (i, j)),
                  pl.BlockSpec((bt, bt), lambda i, j: (j, i))],
        out_specs=pl.BlockSpec((bt, bt), lambda i, j: (i, j)),
        out_shape=jax.ShapeDtypeStruct((D, D), f32),
    )(W, W)

    bm, bn, bk = 256, 256, 512

    def mm(a):
        return pl.pallas_call(
            _mm_kernel,
            grid=(B // bm, D // bn, D // bk),
            in_specs=[pl.BlockSpec((bm, bk), lambda i, j, k: (i, k)),
                      pl.BlockSpec((bk, bn), lambda i, j, k: (k, j))],
            out_specs=pl.BlockSpec((bm, bn), lambda i, j, k: (i, j)),
            out_shape=jax.ShapeDtypeStruct((B, D), f32),
            compiler_params=pltpu.CompilerParams(
                dimension_semantics=("parallel", "parallel", "arbitrary")),
        )(a, Wsym)

    xw = mm(x)

    bb = 256
    nb = B // bb
    row = lambda ib: (ib, 0)
    y, idxbuf, lf = pl.pallas_call(
        _fwd_kernel,
        grid=(nb,),
        in_specs=[
            pl.BlockSpec(memory_space=pltpu.SMEM),
            pl.BlockSpec((bb, D), row),
            pl.BlockSpec((bb, D), row),
            pl.BlockSpec((8, D), lambda ib: (0, 0)),
            pl.BlockSpec((bb, 128), row),
        ],
        out_specs=[
            pl.BlockSpec((bb, D), row),
            pl.BlockSpec((bb, 128), row),
            pl.BlockSpec((bb, 128), row),
        ],
        out_shape=[jax.ShapeDtypeStruct((B, D), f32),
                   jax.ShapeDtypeStruct((B, 128), jnp.int32),
                   jax.ShapeDtypeStruct((B, 128), f32)],
        scratch_shapes=[pltpu.VMEM((bb, D), f32)],
        compiler_params=pltpu.CompilerParams(
            dimension_semantics=("arbitrary",)),
    )(karr, x, xw, bv, rad128)

    yw = mm(y)

    new_x = pl.pallas_call(
        _bwd_kernel,
        grid=(nb,),
        in_specs=[
            pl.BlockSpec((bb, D), row),
            pl.BlockSpec((bb, D), row),
            pl.BlockSpec((bb, D), row),
            pl.BlockSpec((8, D), lambda ib: (0, 0)),
            pl.BlockSpec((bb, 128), row),
            pl.BlockSpec((bb, 128), row),
            pl.BlockSpec((bb, 128), row),
            pl.BlockSpec((bb, 128), row),
        ],
        out_specs=pl.BlockSpec((bb, D), row),
        out_shape=jax.ShapeDtypeStruct((B, D), f32),
        scratch_shapes=[pltpu.VMEM((bb, D), f32)],
        compiler_params=pltpu.CompilerParams(
            dimension_semantics=("arbitrary",)),
    )(x, yw, y, bv, rad128, idxbuf, lf, u128)
    return new_x


# speedup vs baseline: 2.7689x; 1.3812x over previous
import jax
import jax.numpy as jnp
from jax.experimental import pallas as pl
from jax.experimental.pallas import tpu as pltpu

_MAXR = 15


def _mmsym_kernel(a_ref, w1_ref, w2_ref, o_ref):
    o_ref[...] = jnp.dot(a_ref[...], w1_ref[...] + w2_ref[...].T,
                         preferred_element_type=jnp.float32)


def _fwd_kernel(karr_ref, x_ref, xw_ref, bv_ref, rad_ref, y_ref, idx_ref,
                lf_ref, sg_ref):
    ib = pl.program_id(0)
    bb, D = x_ref.shape
    x = x_ref[...]
    bv = bv_ref[0:1, :]
    gh = (xw_ref[...] + bv) * 0.5
    sg0 = (1.0 - 2.0 * x) * gh
    sg_ref[...] = sg0
    s0 = jnp.sum(jnp.exp(sg0), axis=1, keepdims=True)
    sx = jnp.sum(xw_ref[...] * x * 0.5 + x * bv, axis=1, keepdims=True)
    idx_ref[...] = jnp.zeros_like(idx_ref)
    lane = jax.lax.broadcasted_iota(jnp.int32, (bb, D), 1)
    rowi = jax.lax.broadcasted_iota(jnp.int32, (bb, D), 0)
    cnt = (rowi * D + lane + ib * (bb * D)).astype(jnp.uint32)
    li = jax.lax.broadcasted_iota(jnp.int32, idx_ref.shape, 1)
    rad = rad_ref[:, 0:1]
    rots = ((13, 15, 26, 6), (17, 29, 16, 24))

    def body(s, carry):
        ssum, lf = carry
        ks0 = karr_ref[0, s]
        ks1 = karr_ref[1, s]
        ks2 = karr_ref[2, s]
        ksl = (ks0, ks1, ks2)
        h0 = ks0
        h1 = cnt + ks1
        for i in range(5):
            for r in rots[i % 2]:
                h0 = h0 + h1
                h1 = (h1 << jnp.uint32(r)) | (h1 >> jnp.uint32(32 - r))
                h1 = h0 ^ h1
            h0 = h0 + ksl[(i + 1) % 3]
            h1 = h1 + ksl[(i + 2) % 3] + jnp.uint32(i + 1)
        bits = h0 ^ h1
        fb = (bits >> jnp.uint32(9)) | jnp.uint32(0x3F800000)
        f = jax.lax.bitcast_convert_type(fb, jnp.float32) - 1.0
        tiny = jnp.float32(jnp.finfo(jnp.float32).tiny)
        uu = jnp.maximum(tiny, f * (jnp.float32(1.0) - tiny) + tiny)
        g = -jnp.log(-jnp.log(uu))

        sg = sg_ref[...]
        z = sg + g
        zmx = jnp.max(z, axis=1, keepdims=True)
        idx = jnp.min(jnp.where(z == zmx, lane, D), axis=1, keepdims=True)
        onehot = lane == idx
        val = jnp.sum(jnp.where(onehot, sg, 0.0), axis=1, keepdims=True)
        maskb = rad > s
        maskf = maskb.astype(jnp.float32)
        lf = lf + (val - jnp.log(ssum)) * maskf
        flip = onehot & maskb
        sg_ref[...] = jnp.where(flip, -sg, sg)
        ssum = ssum + (jnp.exp(-val) - jnp.exp(val)) * maskf
        idx_ref[...] = jnp.where(li == s, jnp.broadcast_to(idx, idx_ref.shape),
                                 idx_ref[...])
        return ssum, lf

    _, lf = jax.lax.fori_loop(0, _MAXR, body, (s0, sx))
    lf_ref[...] = jnp.broadcast_to(lf, lf_ref.shape)
    sgi = jax.lax.bitcast_convert_type(sg_ref[...], jnp.uint32)
    ghi = jax.lax.bitcast_convert_type((xw_ref[...] + bv) * 0.5, jnp.uint32)
    y_ref[...] = ((sgi ^ ghi) >> jnp.uint32(31)).astype(jnp.float32)


def _bwd_kernel(x_ref, yw_ref, y_ref, bv_ref, rad_ref, idx_ref, lf_ref,
                u_ref, o_ref, sg_ref):
    bb, D = x_ref.shape
    x = x_ref[...]
    y = y_ref[...]
    bv = bv_ref[0:1, :]
    gh = (yw_ref[...] + bv) * 0.5
    sg0 = (1.0 - 2.0 * x) * gh
    sg_ref[...] = sg0
    s0 = jnp.sum(jnp.exp(sg0), axis=1, keepdims=True)
    sy = jnp.sum(yw_ref[...] * y * 0.5 + y * bv, axis=1, keepdims=True)
    idxb = idx_ref[...]
    lane = jax.lax.broadcasted_iota(jnp.int32, (bb, D), 1)
    li = jax.lax.broadcasted_iota(jnp.int32, idxb.shape, 1)
    rad = rad_ref[:, 0:1]

    def body(s, carry):
        ssum, lb = carry
        idx = jnp.sum(jnp.where(li == s, idxb, 0), axis=1, keepdims=True)
        onehot = lane == idx
        maskb = rad > s
        maskf = maskb.astype(jnp.float32)
        sg = sg_ref[...]
        l_old = jnp.sum(jnp.where(onehot, sg, 0.0), axis=1, keepdims=True)
        sg_ref[...] = jnp.where(onehot & maskb, -sg, sg)
        ssum = ssum + (jnp.exp(-l_old) - jnp.exp(l_old)) * maskf
        lb = lb + (-l_old - jnp.log(ssum)) * maskf
        return ssum, lb

    _, lb = jax.lax.fori_loop(0, _MAXR, body, (s0, sy))
    la = lb - lf_ref[:, 0:1]
    acc = (jnp.exp(la) >= u_ref[:, 0:1]).astype(jnp.float32)
    o_ref[...] = y * acc + (1.0 - acc) * x


def kernel(x, W, b):
    B, D = x.shape
    f32 = jnp.float32
    key = jax.random.key(42)
    k_r, k_cat, k_acc = jax.random.split(key, 3)
    radius = jax.random.randint(k_r, (B, 1), 1, 16)
    u = jax.random.uniform(k_acc, (B,), dtype=f32)
    kd = jax.vmap(
        lambda s: jax.random.key_data(jax.random.fold_in(k_cat, s)),
        out_axes=1)(jnp.arange(_MAXR))
    karr = jnp.concatenate(
        [kd, (kd[0:1] ^ kd[1:2] ^ jnp.uint32(0x1BD11BDA))])

    rad128 = jnp.broadcast_to(radius, (B, 128))
    u128 = jnp.broadcast_to(u[:, None], (B, 128))
    bv = jnp.broadcast_to(b[None, :], (8, D))

    bn = 256

    def mm(a):
        return pl.pallas_call(
            _mmsym_kernel,
            grid=(D // bn,),
            in_specs=[pl.BlockSpec((B, D), lambda j: (0, 0)),
                      pl.BlockSpec((D, bn), lambda j: (0, j)),
                      pl.BlockSpec((bn, D), lambda j: (j, 0))],
            out_specs=pl.BlockSpec((B, bn), lambda j: (0, j)),
            out_shape=jax.ShapeDtypeStruct((B, D), f32),
            compiler_params=pltpu.CompilerParams(
                dimension_semantics=("arbitrary",)),
        )(a, W, W)

    xw = mm(x)

    bb = 256
    nb = B // bb
    row = lambda ib: (ib, 0)
    y, idxbuf, lf = pl.pallas_call(
        _fwd_kernel,
        grid=(nb,),
        in_specs=[
            pl.BlockSpec(memory_space=pltpu.SMEM),
            pl.BlockSpec((bb, D), row),
            pl.BlockSpec((bb, D), row),
            pl.BlockSpec((8, D), lambda ib: (0, 0)),
            pl.BlockSpec((bb, 128), row),
        ],
        out_specs=[
            pl.BlockSpec((bb, D), row),
            pl.BlockSpec((bb, 128), row),
            pl.BlockSpec((bb, 128), row),
        ],
        out_shape=[jax.ShapeDtypeStruct((B, D), f32),
                   jax.ShapeDtypeStruct((B, 128), jnp.int32),
                   jax.ShapeDtypeStruct((B, 128), f32)],
        scratch_shapes=[pltpu.VMEM((bb, D), f32)],
        compiler_params=pltpu.CompilerParams(
            dimension_semantics=("arbitrary",)),
    )(karr, x, xw, bv, rad128)

    yw = mm(y)

    new_x = pl.pallas_call(
        _bwd_kernel,
        grid=(nb,),
        in_specs=[
            pl.BlockSpec((bb, D), row),
            pl.BlockSpec((bb, D), row),
            pl.BlockSpec((bb, D), row),
            pl.BlockSpec((8, D), lambda ib: (0, 0)),
            pl.BlockSpec((bb, 128), row),
            pl.BlockSpec((bb, 128), row),
            pl.BlockSpec((bb, 128), row),
            pl.BlockSpec((bb, 128), row),
        ],
        out_specs=pl.BlockSpec((bb, D), row),
        out_shape=jax.ShapeDtypeStruct((B, D), f32),
        scratch_shapes=[pltpu.VMEM((bb, D), f32)],
        compiler_params=pltpu.CompilerParams(
            dimension_semantics=("arbitrary",)),
    )(x, yw, y, bv, rad128, idxbuf, lf, u128)
    return new_x


# speedup vs baseline: 2.7831x; 1.0051x over previous
import jax
import jax.numpy as jnp
from jax.experimental import pallas as pl
from jax.experimental.pallas import tpu as pltpu

_MAXR = 15


def _mmsym_kernel(a_ref, w1_ref, w2_ref, o_ref):
    o_ref[...] = jnp.dot(a_ref[...], w1_ref[...] + w2_ref[...].T,
                         preferred_element_type=jnp.float32)


def _fwd_kernel(karr_ref, x_ref, xw_ref, bv_ref, rad_ref, y_ref, idx_ref,
                lf_ref, sg_ref):
    ib = pl.program_id(0)
    bb, D = x_ref.shape
    x = x_ref[...]
    bv = bv_ref[0:1, :]
    gh = (xw_ref[...] + bv) * 0.5
    sg0 = (1.0 - 2.0 * x) * gh
    sg_ref[...] = sg0
    s0 = jnp.sum(jnp.exp(sg0), axis=1, keepdims=True)
    sx = jnp.sum(xw_ref[...] * x * 0.5 + x * bv, axis=1, keepdims=True)
    idx_ref[...] = jnp.zeros_like(idx_ref)
    lane = jax.lax.broadcasted_iota(jnp.int32, (bb, D), 1)
    rowi = jax.lax.broadcasted_iota(jnp.int32, (bb, D), 0)
    cnt = (rowi * D + lane + ib * (bb * D)).astype(jnp.uint32)
    li = jax.lax.broadcasted_iota(jnp.int32, idx_ref.shape, 1)
    rad = rad_ref[:, 0:1]
    rots = ((13, 15, 26, 6), (17, 29, 16, 24))

    def body(s, carry):
        ssum, lf = carry
        ks0 = karr_ref[0, s]
        ks1 = karr_ref[1, s]
        ks2 = karr_ref[2, s]
        ksl = (ks0, ks1, ks2)
        h0 = ks0
        h1 = cnt + ks1
        for i in range(5):
            for r in rots[i % 2]:
                h0 = h0 + h1
                h1 = (h1 << jnp.uint32(r)) | (h1 >> jnp.uint32(32 - r))
                h1 = h0 ^ h1
            h0 = h0 + ksl[(i + 1) % 3]
            h1 = h1 + ksl[(i + 2) % 3] + jnp.uint32(i + 1)
        bits = h0 ^ h1
        fb = (bits >> jnp.uint32(9)) | jnp.uint32(0x3F800000)
        f = jax.lax.bitcast_convert_type(fb, jnp.float32) - 1.0
        tiny = jnp.float32(jnp.finfo(jnp.float32).tiny)
        uu = jnp.maximum(tiny, f * (jnp.float32(1.0) - tiny) + tiny)
        g = -jnp.log(-jnp.log(uu))

        sg = sg_ref[...]
        z = sg + g
        zmx = jnp.max(z, axis=1, keepdims=True)
        idx = jnp.min(jnp.where(z == zmx, lane, D), axis=1, keepdims=True)
        onehot = lane == idx
        val = jnp.sum(jnp.where(onehot, sg, 0.0), axis=1, keepdims=True)
        maskb = rad > s
        maskf = maskb.astype(jnp.float32)
        lf = lf + (val - jnp.log(ssum)) * maskf
        flip = onehot & maskb
        sg_ref[...] = jnp.where(flip, -sg, sg)
        ssum = ssum + (jnp.exp(-val) - jnp.exp(val)) * maskf
        idx_ref[...] = jnp.where(li == s, jnp.broadcast_to(idx, idx_ref.shape),
                                 idx_ref[...])
        return ssum, lf

    _, lf = jax.lax.fori_loop(0, _MAXR, body, (s0, sx))
    lf_ref[...] = jnp.broadcast_to(lf, lf_ref.shape)
    sgi = jax.lax.bitcast_convert_type(sg_ref[...], jnp.uint32)
    ghi = jax.lax.bitcast_convert_type((xw_ref[...] + bv) * 0.5, jnp.uint32)
    y_ref[...] = ((sgi ^ ghi) >> jnp.uint32(31)).astype(jnp.float32)


def _bwd_kernel(x_ref, yw_ref, y_ref, bv_ref, rad_ref, idx_ref, lf_ref,
                u_ref, o_ref, sg_ref):
    bb, D = x_ref.shape
    x = x_ref[...]
    y = y_ref[...]
    bv = bv_ref[0:1, :]
    gh = (yw_ref[...] + bv) * 0.5
    sg0 = (1.0 - 2.0 * x) * gh
    sg_ref[...] = sg0
    s0 = jnp.sum(jnp.exp(sg0), axis=1, keepdims=True)
    sy = jnp.sum(yw_ref[...] * y * 0.5 + y * bv, axis=1, keepdims=True)
    idxb = idx_ref[...]
    lane = jax.lax.broadcasted_iota(jnp.int32, (bb, D), 1)
    li = jax.lax.broadcasted_iota(jnp.int32, idxb.shape, 1)
    rad = rad_ref[:, 0:1]

    def body(s, carry):
        ssum, lb = carry
        idx = jnp.sum(jnp.where(li == s, idxb, 0), axis=1, keepdims=True)
        onehot = lane == idx
        maskb = rad > s
        maskf = maskb.astype(jnp.float32)
        sg = sg_ref[...]
        l_old = jnp.sum(jnp.where(onehot, sg, 0.0), axis=1, keepdims=True)
        sg_ref[...] = jnp.where(onehot & maskb, -sg, sg)
        ssum = ssum + (jnp.exp(-l_old) - jnp.exp(l_old)) * maskf
        lb = lb + (-l_old - jnp.log(ssum)) * maskf
        return ssum, lb

    _, lb = jax.lax.fori_loop(0, _MAXR, body, (s0, sy))
    la = lb - lf_ref[:, 0:1]
    acc = (jnp.exp(la) >= u_ref[:, 0:1]).astype(jnp.float32)
    o_ref[...] = y * acc + (1.0 - acc) * x


def kernel(x, W, b):
    B, D = x.shape
    f32 = jnp.float32
    key = jax.random.key(42)
    k_r, k_cat, k_acc = jax.random.split(key, 3)
    radius = jax.random.randint(k_r, (B, 1), 1, 16)
    u = jax.random.uniform(k_acc, (B,), dtype=f32)
    kd = jax.vmap(
        lambda s: jax.random.key_data(jax.random.fold_in(k_cat, s)),
        out_axes=1)(jnp.arange(_MAXR))
    karr = jnp.concatenate(
        [kd, (kd[0:1] ^ kd[1:2] ^ jnp.uint32(0x1BD11BDA))])

    rad128 = jnp.broadcast_to(radius, (B, 128))
    u128 = jnp.broadcast_to(u[:, None], (B, 128))
    bv = jnp.broadcast_to(b[None, :], (8, D))

    bn = 256

    def mm(a):
        return pl.pallas_call(
            _mmsym_kernel,
            grid=(D // bn,),
            in_specs=[pl.BlockSpec((B, D), lambda j: (0, 0)),
                      pl.BlockSpec((D, bn), lambda j: (0, j)),
                      pl.BlockSpec((bn, D), lambda j: (j, 0))],
            out_specs=pl.BlockSpec((B, bn), lambda j: (0, j)),
            out_shape=jax.ShapeDtypeStruct((B, D), f32),
            compiler_params=pltpu.CompilerParams(
                dimension_semantics=("arbitrary",)),
        )(a, W, W)

    xw = mm(x)

    bb = 512
    nb = B // bb
    row = lambda ib: (ib, 0)
    y, idxbuf, lf = pl.pallas_call(
        _fwd_kernel,
        grid=(nb,),
        in_specs=[
            pl.BlockSpec(memory_space=pltpu.SMEM),
            pl.BlockSpec((bb, D), row),
            pl.BlockSpec((bb, D), row),
            pl.BlockSpec((8, D), lambda ib: (0, 0)),
            pl.BlockSpec((bb, 128), row),
        ],
        out_specs=[
            pl.BlockSpec((bb, D), row),
            pl.BlockSpec((bb, 128), row),
            pl.BlockSpec((bb, 128), row),
        ],
        out_shape=[jax.ShapeDtypeStruct((B, D), f32),
                   jax.ShapeDtypeStruct((B, 128), jnp.int32),
                   jax.ShapeDtypeStruct((B, 128), f32)],
        scratch_shapes=[pltpu.VMEM((bb, D), f32)],
        compiler_params=pltpu.CompilerParams(
            dimension_semantics=("arbitrary",)),
    )(karr, x, xw, bv, rad128)

    yw = mm(y)

    new_x = pl.pallas_call(
        _bwd_kernel,
        grid=(nb,),
        in_specs=[
            pl.BlockSpec((bb, D), row),
            pl.BlockSpec((bb, D), row),
            pl.BlockSpec((bb, D), row),
            pl.BlockSpec((8, D), lambda ib: (0, 0)),
            pl.BlockSpec((bb, 128), row),
            pl.BlockSpec((bb, 128), row),
            pl.BlockSpec((bb, 128), row),
            pl.BlockSpec((bb, 128), row),
        ],
        out_specs=pl.BlockSpec((bb, D), row),
        out_shape=jax.ShapeDtypeStruct((B, D), f32),
        scratch_shapes=[pltpu.VMEM((bb, D), f32)],
        compiler_params=pltpu.CompilerParams(
            dimension_semantics=("arbitrary",)),
    )(x, yw, y, bv, rad128, idxbuf, lf, u128)
    return new_x


# speedup vs baseline: 2.7997x; 1.0060x over previous
import jax
import jax.numpy as jnp
from jax.experimental import pallas as pl
from jax.experimental.pallas import tpu as pltpu

_MAXR = 15


def _mmsym_kernel(a_ref, w1_ref, w2_ref, o_ref):
    o_ref[...] = jnp.dot(a_ref[...], w1_ref[...] + w2_ref[...].T,
                         preferred_element_type=jnp.float32)


def _fwd_kernel(karr_ref, x_ref, w_ref, bv_ref, rad_ref, y_ref, idx_ref,
                lf_ref, sg_ref, xw_ref):
    ib = pl.program_id(0)
    bb, D = x_ref.shape
    x = x_ref[...]
    bv = bv_ref[0:1, :]
    xw_ref[...] = (jnp.dot(x, w_ref[...], preferred_element_type=jnp.float32)
                   + jax.lax.dot_general(
                       x, w_ref[...], (((1,), (1,)), ((), ())),
                       preferred_element_type=jnp.float32))
    gh = (xw_ref[...] + bv) * 0.5
    sg0 = (1.0 - 2.0 * x) * gh
    sg_ref[...] = sg0
    s0 = jnp.sum(jnp.exp(sg0), axis=1, keepdims=True)
    sx = jnp.sum(xw_ref[...] * x * 0.5 + x * bv, axis=1, keepdims=True)
    idx_ref[...] = jnp.zeros_like(idx_ref)
    lane = jax.lax.broadcasted_iota(jnp.int32, (bb, D), 1)
    rowi = jax.lax.broadcasted_iota(jnp.int32, (bb, D), 0)
    cnt = (rowi * D + lane + ib * (bb * D)).astype(jnp.uint32)
    li = jax.lax.broadcasted_iota(jnp.int32, idx_ref.shape, 1)
    rad = rad_ref[:, 0:1]
    rots = ((13, 15, 26, 6), (17, 29, 16, 24))

    def body(s, carry):
        ssum, lf = carry
        ks0 = karr_ref[0, s]
        ks1 = karr_ref[1, s]
        ks2 = karr_ref[2, s]
        ksl = (ks0, ks1, ks2)
        h0 = ks0
        h1 = cnt + ks1
        for i in range(5):
            for r in rots[i % 2]:
                h0 = h0 + h1
                h1 = (h1 << jnp.uint32(r)) | (h1 >> jnp.uint32(32 - r))
                h1 = h0 ^ h1
            h0 = h0 + ksl[(i + 1) % 3]
            h1 = h1 + ksl[(i + 2) % 3] + jnp.uint32(i + 1)
        bits = h0 ^ h1
        fb = (bits >> jnp.uint32(9)) | jnp.uint32(0x3F800000)
        f = jax.lax.bitcast_convert_type(fb, jnp.float32) - 1.0
        tiny = jnp.float32(jnp.finfo(jnp.float32).tiny)
        uu = jnp.maximum(tiny, f * (jnp.float32(1.0) - tiny) + tiny)
        g = -jnp.log(-jnp.log(uu))

        sg = sg_ref[...]
        z = sg + g
        zmx = jnp.max(z, axis=1, keepdims=True)
        idx = jnp.min(jnp.where(z == zmx, lane, D), axis=1, keepdims=True)
        onehot = lane == idx
        val = jnp.sum(jnp.where(onehot, sg, 0.0), axis=1, keepdims=True)
        maskb = rad > s
        maskf = maskb.astype(jnp.float32)
        lf = lf + (val - jnp.log(ssum)) * maskf
        flip = onehot & maskb
        sg_ref[...] = jnp.where(flip, -sg, sg)
        ssum = ssum + (jnp.exp(-val) - jnp.exp(val)) * maskf
        idx_ref[...] = jnp.where(li == s, jnp.broadcast_to(idx, idx_ref.shape),
                                 idx_ref[...])
        return ssum, lf

    _, lf = jax.lax.fori_loop(0, _MAXR, body, (s0, sx))
    lf_ref[...] = jnp.broadcast_to(lf, lf_ref.shape)
    sgi = jax.lax.bitcast_convert_type(sg_ref[...], jnp.uint32)
    ghi = jax.lax.bitcast_convert_type((xw_ref[...] + bv) * 0.5, jnp.uint32)
    y_ref[...] = ((sgi ^ ghi) >> jnp.uint32(31)).astype(jnp.float32)


def _bwd_kernel(x_ref, w_ref, y_ref, bv_ref, rad_ref, idx_ref, lf_ref,
                u_ref, o_ref, sg_ref, yw_ref):
    bb, D = x_ref.shape
    x = x_ref[...]
    y = y_ref[...]
    bv = bv_ref[0:1, :]
    yw_ref[...] = (jnp.dot(y, w_ref[...], preferred_element_type=jnp.float32)
                   + jax.lax.dot_general(
                       y, w_ref[...], (((1,), (1,)), ((), ())),
                       preferred_element_type=jnp.float32))
    gh = (yw_ref[...] + bv) * 0.5
    sg0 = (1.0 - 2.0 * x) * gh
    sg_ref[...] = sg0
    s0 = jnp.sum(jnp.exp(sg0), axis=1, keepdims=True)
    sy = jnp.sum(yw_ref[...] * y * 0.5 + y * bv, axis=1, keepdims=True)
    idxb = idx_ref[...]
    lane = jax.lax.broadcasted_iota(jnp.int32, (bb, D), 1)
    li = jax.lax.broadcasted_iota(jnp.int32, idxb.shape, 1)
    rad = rad_ref[:, 0:1]

    def body(s, carry):
        ssum, lb = carry
        idx = jnp.sum(jnp.where(li == s, idxb, 0), axis=1, keepdims=True)
        onehot = lane == idx
        maskb = rad > s
        maskf = maskb.astype(jnp.float32)
        sg = sg_ref[...]
        l_old = jnp.sum(jnp.where(onehot, sg, 0.0), axis=1, keepdims=True)
        sg_ref[...] = jnp.where(onehot & maskb, -sg, sg)
        ssum = ssum + (jnp.exp(-l_old) - jnp.exp(l_old)) * maskf
        lb = lb + (-l_old - jnp.log(ssum)) * maskf
        return ssum, lb

    _, lb = jax.lax.fori_loop(0, _MAXR, body, (s0, sy))
    la = lb - lf_ref[:, 0:1]
    acc = (jnp.exp(la) >= u_ref[:, 0:1]).astype(jnp.float32)
    o_ref[...] = y * acc + (1.0 - acc) * x


def kernel(x, W, b):
    B, D = x.shape
    f32 = jnp.float32
    key = jax.random.key(42)
    k_r, k_cat, k_acc = jax.random.split(key, 3)
    radius = jax.random.randint(k_r, (B, 1), 1, 16)
    u = jax.random.uniform(k_acc, (B,), dtype=f32)
    kd = jax.vmap(
        lambda s: jax.random.key_data(jax.random.fold_in(k_cat, s)),
        out_axes=1)(jnp.arange(_MAXR))
    karr = jnp.concatenate(
        [kd, (kd[0:1] ^ kd[1:2] ^ jnp.uint32(0x1BD11BDA))])

    rad128 = jnp.broadcast_to(radius, (B, 128))
    u128 = jnp.broadcast_to(u[:, None], (B, 128))
    bv = jnp.broadcast_to(b[None, :], (8, D))

    bb = 512
    nb = B // bb
    row = lambda ib: (ib, 0)
    full = lambda ib: (0, 0)
    y, idxbuf, lf = pl.pallas_call(
        _fwd_kernel,
        grid=(nb,),
        in_specs=[
            pl.BlockSpec(memory_space=pltpu.SMEM),
            pl.BlockSpec((bb, D), row),
            pl.BlockSpec((D, D), full),
            pl.BlockSpec((8, D), full),
            pl.BlockSpec((bb, 128), row),
        ],
        out_specs=[
            pl.BlockSpec((bb, D), row),
            pl.BlockSpec((bb, 128), row),
            pl.BlockSpec((bb, 128), row),
        ],
        out_shape=[jax.ShapeDtypeStruct((B, D), f32),
                   jax.ShapeDtypeStruct((B, 128), jnp.int32),
                   jax.ShapeDtypeStruct((B, 128), f32)],
        scratch_shapes=[pltpu.VMEM((bb, D), f32),
                        pltpu.VMEM((bb, D), f32)],
        compiler_params=pltpu.CompilerParams(
            dimension_semantics=("arbitrary",)),
    )(karr, x, W, bv, rad128)

    bc = 256
    nc = B // bc
    new_x = pl.pallas_call(
        _bwd_kernel,
        grid=(nc,),
        in_specs=[
            pl.BlockSpec((bc, D), row),
            pl.BlockSpec((D, D), full),
            pl.BlockSpec((bc, D), row),
            pl.BlockSpec((8, D), full),
            pl.BlockSpec((bc, 128), row),
            pl.BlockSpec((bc, 128), row),
            pl.BlockSpec((bc, 128), row),
            pl.BlockSpec((bc, 128), row),
        ],
        out_specs=pl.BlockSpec((bc, D), row),
        out_shape=jax.ShapeDtypeStruct((B, D), f32),
        scratch_shapes=[pltpu.VMEM((bc, D), f32),
                        pltpu.VMEM((bc, D), f32)],
        compiler_params=pltpu.CompilerParams(
            dimension_semantics=("arbitrary",)),
    )(x, W, y, bv, rad128, idxbuf, lf, u128)
    return new_x


# speedup vs baseline: 2.8149x; 1.0054x over previous
import jax
import jax.numpy as jnp
from jax.experimental import pallas as pl
from jax.experimental.pallas import tpu as pltpu

_MAXR = 15


def _fwd_kernel(karr_ref, x_ref, w_ref, bv_ref, rad_ref, y_ref, idx_ref,
                lf_ref, sg_ref, xw_ref):
    ib = pl.program_id(0)
    bb, D = x_ref.shape
    x = x_ref[...]
    bv = bv_ref[0:1, :]
    xw_ref[...] = (jnp.dot(x, w_ref[...], preferred_element_type=jnp.float32)
                   + jax.lax.dot_general(
                       x, w_ref[...], (((1,), (1,)), ((), ())),
                       preferred_element_type=jnp.float32))
    gh = (xw_ref[...] + bv) * 0.5
    sg0 = (1.0 - 2.0 * x) * gh
    sg_ref[...] = sg0
    s0 = jnp.sum(jnp.exp(sg0), axis=1, keepdims=True)
    sx = jnp.sum(xw_ref[...] * x * 0.5 + x * bv, axis=1, keepdims=True)
    idx_ref[...] = jnp.zeros_like(idx_ref)
    lane = jax.lax.broadcasted_iota(jnp.int32, (bb, D), 1)
    rowi = jax.lax.broadcasted_iota(jnp.int32, (bb, D), 0)
    cnt = (rowi * D + lane + ib * (bb * D)).astype(jnp.uint32)
    li = jax.lax.broadcasted_iota(jnp.int32, idx_ref.shape, 1)
    rad = rad_ref[:, 0:1]
    rots = ((13, 15, 26, 6), (17, 29, 16, 24))

    def body(s, carry):
        ssum, lf = carry
        ks0 = karr_ref[0, s]
        ks1 = karr_ref[1, s]
        ks2 = karr_ref[2, s]
        ksl = (ks0, ks1, ks2)
        h0 = ks0
        h1 = cnt + ks1
        for i in range(5):
            for r in rots[i % 2]:
                h0 = h0 + h1
                h1 = (h1 << jnp.uint32(r)) | (h1 >> jnp.uint32(32 - r))
                h1 = h0 ^ h1
            h0 = h0 + ksl[(i + 1) % 3]
            h1 = h1 + ksl[(i + 2) % 3] + jnp.uint32(i + 1)
        bits = h0 ^ h1
        fb = (bits >> jnp.uint32(9)) | jnp.uint32(0x3F800000)
        f = jax.lax.bitcast_convert_type(fb, jnp.float32) - 1.0
        uu = f + jnp.float32(jnp.finfo(jnp.float32).tiny)
        g = -jnp.log(-jnp.log(uu))

        sg = sg_ref[...]
        z = sg + g
        zmx = jnp.max(z, axis=1, keepdims=True)
        idx = jnp.min(jnp.where(z == zmx, lane, D), axis=1, keepdims=True)
        onehot = lane == idx
        val = jnp.sum(jnp.where(onehot, sg, 0.0), axis=1, keepdims=True)
        maskb = rad > s
        maskf = maskb.astype(jnp.float32)
        lf = lf + (val - jnp.log(ssum)) * maskf
        flip = onehot & maskb
        sg_ref[...] = jnp.where(flip, -sg, sg)
        ssum = ssum + (jnp.exp(-val) - jnp.exp(val)) * maskf
        idx_ref[...] = jnp.where(li == s, jnp.broadcast_to(idx, idx_ref.shape),
                                 idx_ref[...])
        return ssum, lf

    _, lf = jax.lax.fori_loop(0, _MAXR, body, (s0, sx))
    lf_ref[...] = jnp.broadcast_to(lf, lf_ref.shape)
    sgi = jax.lax.bitcast_convert_type(sg_ref[...], jnp.uint32)
    ghi = jax.lax.bitcast_convert_type((xw_ref[...] + bv) * 0.5, jnp.uint32)
    y_ref[...] = ((sgi ^ ghi) >> jnp.uint32(31)).astype(jnp.float32)


def _bwd_kernel(x_ref, w_ref, y_ref, bv_ref, rad_ref, idx_ref, lf_ref,
                u_ref, o_ref, sg_ref, yw_ref):
    bb, D = x_ref.shape
    x = x_ref[...]
    y = y_ref[...]
    bv = bv_ref[0:1, :]
    yw_ref[...] = (jnp.dot(y, w_ref[...], preferred_element_type=jnp.float32)
                   + jax.lax.dot_general(
                       y, w_ref[...], (((1,), (1,)), ((), ())),
                       preferred_element_type=jnp.float32))
    gh = (yw_ref[...] + bv) * 0.5
    sg0 = (1.0 - 2.0 * x) * gh
    sg_ref[...] = sg0
    s0 = jnp.sum(jnp.exp(sg0), axis=1, keepdims=True)
    sy = jnp.sum(yw_ref[...] * y * 0.5 + y * bv, axis=1, keepdims=True)
    idxb = idx_ref[...]
    lane = jax.lax.broadcasted_iota(jnp.int32, (bb, D), 1)
    li = jax.lax.broadcasted_iota(jnp.int32, idxb.shape, 1)
    rad = rad_ref[:, 0:1]

    def body(s, carry):
        ssum, lb = carry
        idx = jnp.sum(jnp.where(li == s, idxb, 0), axis=1, keepdims=True)
        onehot = lane == idx
        maskb = rad > s
        maskf = maskb.astype(jnp.float32)
        sg = sg_ref[...]
        l_old = jnp.sum(jnp.where(onehot, sg, 0.0), axis=1, keepdims=True)
        sg_ref[...] = jnp.where(onehot & maskb, -sg, sg)
        ssum = ssum + (jnp.exp(-l_old) - jnp.exp(l_old)) * maskf
        lb = lb + (-l_old - jnp.log(ssum)) * maskf
        return ssum, lb

    _, lb = jax.lax.fori_loop(0, _MAXR, body, (s0, sy))
    la = lb - lf_ref[:, 0:1]
    acc = (jnp.exp(la) >= u_ref[:, 0:1]).astype(jnp.float32)
    o_ref[...] = y * acc + (1.0 - acc) * x


def kernel(x, W, b):
    B, D = x.shape
    f32 = jnp.float32
    key = jax.random.key(42)
    k_r, k_cat, k_acc = jax.random.split(key, 3)
    radius = jax.random.randint(k_r, (B, 1), 1, 16)
    u = jax.random.uniform(k_acc, (B,), dtype=f32)
    kd = jax.vmap(
        lambda s: jax.random.key_data(jax.random.fold_in(k_cat, s)),
        out_axes=1)(jnp.arange(_MAXR))
    karr = jnp.concatenate(
        [kd, (kd[0:1] ^ kd[1:2] ^ jnp.uint32(0x1BD11BDA))])

    rad128 = jnp.broadcast_to(radius, (B, 128))
    u128 = jnp.broadcast_to(u[:, None], (B, 128))
    bv = jnp.broadcast_to(b[None, :], (8, D))

    bb = 512
    nb = B // bb
    row = lambda ib: (ib, 0)
    full = lambda ib: (0, 0)
    y, idxbuf, lf = pl.pallas_call(
        _fwd_kernel,
        grid=(nb,),
        in_specs=[
            pl.BlockSpec(memory_space=pltpu.SMEM),
            pl.BlockSpec((bb, D), row),
            pl.BlockSpec((D, D), full),
            pl.BlockSpec((8, D), full),
            pl.BlockSpec((bb, 128), row),
        ],
        out_specs=[
            pl.BlockSpec((bb, D), row),
            pl.BlockSpec((bb, 128), row),
            pl.BlockSpec((bb, 128), row),
        ],
        out_shape=[jax.ShapeDtypeStruct((B, D), f32),
                   jax.ShapeDtypeStruct((B, 128), jnp.int32),
                   jax.ShapeDtypeStruct((B, 128), f32)],
        scratch_shapes=[pltpu.VMEM((bb, D), f32),
                        pltpu.VMEM((bb, D), f32)],
        compiler_params=pltpu.CompilerParams(
            dimension_semantics=("arbitrary",)),
    )(karr, x, W, bv, rad128)

    bc = 256
    nc = B // bc
    new_x = pl.pallas_call(
        _bwd_kernel,
        grid=(nc,),
        in_specs=[
            pl.BlockSpec((bc, D), row),
            pl.BlockSpec((D, D), full),
            pl.BlockSpec((bc, D), row),
            pl.BlockSpec((8, D), full),
            pl.BlockSpec((bc, 128), row),
            pl.BlockSpec((bc, 128), row),
            pl.BlockSpec((bc, 128), row),
            pl.BlockSpec((bc, 128), row),
        ],
        out_specs=pl.BlockSpec((bc, D), row),
        out_shape=jax.ShapeDtypeStruct((B, D), f32),
        scratch_shapes=[pltpu.VMEM((bc, D), f32),
                        pltpu.VMEM((bc, D), f32)],
        compiler_params=pltpu.CompilerParams(
            dimension_semantics=("arbitrary",)),
    )(x, W, y, bv, rad128, idxbuf, lf, u128)
    return new_x
